# Initial kernel scaffold; baseline (speedup 1.0000x reference)
import jax
import jax.numpy as jnp
from jax.experimental import pallas as pl


def kernel(x, edge_index, W1, b1, W2, b2):
    raise NotImplementedError("write your pallas kernel here")


# speedup vs baseline: 2.8711x; 2.8711x over previous
import functools

import jax
import jax.numpy as jnp
from jax import lax
from jax.experimental import pallas as pl
from jax.experimental.pallas import tpu as pltpu
from jax.experimental.pallas import tpu_sc as plsc

N_NODES = 10000
N_EDGES = 320000
D = 128
HID = 64

NC = 2
NS = 16
NW = NC * NS
EPW = N_EDGES // NW
CHUNK = 80
NCHUNKS = EPW // CHUNK

_mesh = plsc.VectorSubcoreMesh(core_axis_name="c", subcore_axis_name="s")


@functools.partial(
    pl.kernel,
    mesh=_mesh,
    out_type=jax.ShapeDtypeStruct((N_EDGES, D), jnp.float32),
    scratch_types=[
        pltpu.VMEM((EPW,), jnp.int32),
        pltpu.VMEM((EPW,), jnp.int32),
        pltpu.VMEM((CHUNK, D), jnp.float32),
        pltpu.VMEM((CHUNK, D), jnp.float32),
        pltpu.SemaphoreType.DMA,
        pltpu.SemaphoreType.DMA,
    ],
)
def _gather_absdiff(x_hbm, src_hbm, dst_hbm, out_hbm,
                    idx_s, idx_d, buf_a, buf_b, sem_a, sem_b):
    wid = lax.axis_index("s") * NC + lax.axis_index("c")
    base0 = wid * EPW
    pltpu.sync_copy(src_hbm.at[pl.ds(base0, EPW)], idx_s)
    pltpu.sync_copy(dst_hbm.at[pl.ds(base0, EPW)], idx_d)

    def chunk_body(i, carry):
        off = i * CHUNK
        cp_a = pltpu.async_copy(x_hbm.at[idx_s.at[pl.ds(off, CHUNK)]], buf_a, sem_a)
        cp_b = pltpu.async_copy(x_hbm.at[idx_d.at[pl.ds(off, CHUNK)]], buf_b, sem_b)
        cp_a.wait()
        cp_b.wait()

        def row_body(r, c2):
            for c in range(D // 16):
                sl = pl.ds(c * 16, 16)
                buf_a[r, sl] = jnp.abs(buf_a[r, sl] - buf_b[r, sl])
            return c2

        lax.fori_loop(0, CHUNK, row_body, 0)
        pltpu.sync_copy(buf_a, out_hbm.at[pl.ds(base0 + off, CHUNK)])
        return carry

    lax.fori_loop(0, NCHUNKS, chunk_body, 0)


BLK = 2560
NB = N_EDGES // BLK


def _mlp_body(d_ref, w1_ref, b1_ref, w2_ref, b2_ref, o_ref):
    h = jnp.dot(d_ref[...], w1_ref[...], preferred_element_type=jnp.float32)
    h = jnp.maximum(h + b1_ref[...], 0.0)
    logits = jnp.sum(h * w2_ref[...], axis=1) + b2_ref[0, 0]
    o_ref[...] = jax.nn.sigmoid(logits).reshape(1, 1, BLK)


def _mlp(diff, w1, b1r, w2r, b2r):
    return pl.pallas_call(
        _mlp_body,
        grid=(NB,),
        in_specs=[
            pl.BlockSpec((BLK, D), lambda g: (g, 0)),
            pl.BlockSpec((D, HID), lambda g: (0, 0)),
            pl.BlockSpec((1, HID), lambda g: (0, 0)),
            pl.BlockSpec((1, HID), lambda g: (0, 0)),
            pl.BlockSpec((1, 1), lambda g: (0, 0)),
        ],
        out_specs=pl.BlockSpec((1, 1, BLK), lambda g: (g, 0, 0)),
        out_shape=jax.ShapeDtypeStruct((NB, 1, BLK), jnp.float32),
    )(diff, w1, b1r, w2r, b2r)


def kernel(x, edge_index, W1, b1, W2, b2):
    ei = edge_index
    src = ei[0]
    dst = ei[1]
    diff = _gather_absdiff(x, src, dst)
    vals = _mlp(diff, W1, b1.reshape(1, HID), W2.reshape(1, HID),
                b2.reshape(1, 1))
    values = vals.reshape(-1)
    return (ei, values)
